# Initial kernel scaffold; baseline (speedup 1.0000x reference)
import jax
import jax.numpy as jnp
from jax.experimental import pallas as pl


def kernel(input, adj, W):
    raise NotImplementedError("write your pallas kernel here")


# speedup vs baseline: 1.0001x; 1.0001x over previous
import jax
import jax.numpy as jnp
from jax.experimental import pallas as pl


def _support_kernel(x_ref, w_ref, out_ref):
    out_ref[...] = jnp.dot(
        x_ref[...].astype(jnp.bfloat16),
        w_ref[...].astype(jnp.bfloat16),
        preferred_element_type=jnp.float32,
    ).astype(jnp.bfloat16)


def _spmm_kernel(adj_ref, s_ref, out_ref):
    acc = jnp.dot(
        adj_ref[...].astype(jnp.bfloat16),
        s_ref[...],
        preferred_element_type=jnp.float32,
    )
    out_ref[...] = jnp.maximum(acc, 0.0)


def kernel(input, adj, W):
    n, d_in = input.shape
    d_out = W.shape[1]

    support = pl.pallas_call(
        _support_kernel,
        out_shape=jax.ShapeDtypeStruct((n, d_out), jnp.bfloat16),
    )(input, W)

    bm = 400
    out = pl.pallas_call(
        _spmm_kernel,
        grid=(n // bm,),
        in_specs=[
            pl.BlockSpec((bm, n), lambda i: (i, 0)),
            pl.BlockSpec((n, d_out), lambda i: (0, 0)),
        ],
        out_specs=pl.BlockSpec((bm, d_out), lambda i: (i, 0)),
        out_shape=jax.ShapeDtypeStruct((n, d_out), jnp.float32),
    )(adj, support)
    return out
